# baseline (speedup 1.0000x reference)
import functools

import jax
import jax.numpy as jnp
from jax import lax
from jax.experimental import pallas as pl
from jax.experimental.pallas import tpu as pltpu
from jax.experimental.pallas import tpu_sc as plsc

N_ATOMS = 50000
N_PAIRS = 800000
N_ATOMBASIS = 128
N_RADIALBASIS = 64
N_MAXATOM = 94
CUTOFF = 5.0

_SC_CHUNK = 1256
_SC_PAD = 1264
_TC_L = 16384
_TC_R = _TC_L // 128


def _sc_pair_gather(pos8, idx_i, idx_j, pair_lo, n_pairs):
    info = plsc.get_sparse_core_info()
    nc, ns = info.num_cores, info.num_subcores
    nw = nc * ns
    per_w = n_pairs // nw
    offs, szs = [], []
    o = 0
    while o < per_w:
        sz = min(_SC_CHUNK, per_w - o)
        offs.append(o)
        szs.append(sz)
        o += sz
    n_ch = len(offs)

    mesh = plsc.VectorSubcoreMesh(core_axis_name="c", subcore_axis_name="s")

    @functools.partial(
        pl.kernel,
        out_type=tuple(
            jax.ShapeDtypeStruct((n_pairs,), jnp.float32) for _ in range(3)
        ),
        mesh=mesh,
        scratch_types=[
            pltpu.VMEM((per_w,), jnp.int32),
            pltpu.VMEM((per_w,), jnp.int32),
            [pltpu.VMEM((_SC_PAD, 8), jnp.float32) for _ in range(2)],
            [pltpu.VMEM((_SC_PAD, 8), jnp.float32) for _ in range(2)],
            [pltpu.VMEM((_SC_PAD,), jnp.float32) for _ in range(2)],
            [pltpu.VMEM((_SC_PAD,), jnp.float32) for _ in range(2)],
            [pltpu.VMEM((_SC_PAD,), jnp.float32) for _ in range(2)],
            [pltpu.SemaphoreType.DMA for _ in range(2)],
            [pltpu.SemaphoreType.DMA for _ in range(2)],
            [pltpu.SemaphoreType.DMA for _ in range(2)],
        ],
        compiler_params=pltpu.CompilerParams(use_tc_tiling_on_sc=False,
                                             needs_layout_passes=False),
    )
    def gather_kernel(pos_hbm, ii_hbm, jj_hbm,
                      ox_hbm, oy_hbm, oz_hbm,
                      ii_v, jj_v, pi_v, pj_v,
                      dfx_v, dfy_v, dfz_v, sem_i, sem_j, sem_w):
        wid = lax.axis_index("s") * nc + lax.axis_index("c")
        lane = lax.iota(jnp.int32, 16)
        wbase = wid * per_w

        pltpu.sync_copy(ii_hbm.at[pl.ds(pair_lo + wbase, per_w)], ii_v)
        pltpu.sync_copy(jj_hbm.at[pl.ds(pair_lo + wbase, per_w)], jj_v)

        def run_compute(b, n_grp):
            @plsc.parallel_loop(0, n_grp * 16, 16, unroll=4)
            def _loop(base16):
                rows = base16 + lane
                c0 = jnp.zeros((16,), jnp.int32)
                ax = (plsc.load_gather(pj_v[b], [rows, c0])
                      - plsc.load_gather(pi_v[b], [rows, c0]))
                ay = (plsc.load_gather(pj_v[b], [rows, c0 + 1])
                      - plsc.load_gather(pi_v[b], [rows, c0 + 1]))
                az = (plsc.load_gather(pj_v[b], [rows, c0 + 2])
                      - plsc.load_gather(pi_v[b], [rows, c0 + 2]))
                dfx_v[b][pl.ds(base16, 16)] = ax
                dfy_v[b][pl.ds(base16, 16)] = ay
                dfz_v[b][pl.ds(base16, 16)] = az

        def start_chunk(t, b):
            sz = szs[t]
            cp_i = pltpu.async_copy(
                pos_hbm.at[ii_v.at[pl.ds(offs[t], sz)]],
                pi_v[b].at[pl.ds(0, sz)], sem_i[b])
            cp_j = pltpu.async_copy(
                pos_hbm.at[jj_v.at[pl.ds(offs[t], sz)]],
                pj_v[b].at[pl.ds(0, sz)], sem_j[b])
            return cp_i, cp_j

        cps = [start_chunk(0, 0),
               start_chunk(1, 1) if n_ch > 1 else None]
        wb = [None, None]
        for t in range(n_ch):
            b = t % 2
            cps[b][0].wait()
            cps[b][1].wait()
            if wb[b] is not None:
                for c in wb[b]:
                    c.wait()
            run_compute(b, (szs[t] + 15) // 16)
            base = wbase + offs[t]
            sz = szs[t]
            wb[b] = [
                pltpu.async_copy(src.at[pl.ds(0, sz)],
                                 dst.at[pl.ds(base, sz)], sem_w[b])
                for src, dst in ((dfx_v[b], ox_hbm), (dfy_v[b], oy_hbm),
                                 (dfz_v[b], oz_hbm))
            ]
            if t + 2 < n_ch:
                cps[b] = start_chunk(t + 2, b)
        for b in (0, 1):
            if wb[b] is not None:
                for c in wb[b]:
                    c.wait()

    return gather_kernel(pos8, idx_i, idx_j)


def _tc_embed(an2d, table_pad):
    blk = 2000

    def body(an_ref, tab_ref, out_ref):
        an = an_ref[...]
        oh = (lax.broadcasted_iota(jnp.int32, (blk, 128), 1) == an)
        emb = jnp.dot(oh.astype(jnp.float32), tab_ref[...],
                      preferred_element_type=jnp.float32)
        norm = jnp.sqrt(jnp.sum(emb * emb, axis=-1, keepdims=True))
        scale = jnp.minimum(1.0, float(N_ATOMBASIS) / (norm + 1e-7))
        out_ref[...] = emb * scale

    return pl.pallas_call(
        body,
        grid=(N_ATOMS // blk,),
        in_specs=[
            pl.BlockSpec((blk, 1), lambda i: (i, 0)),
            pl.BlockSpec((128, 128), lambda i: (0, 0)),
        ],
        out_specs=pl.BlockSpec((blk, 128), lambda i: (i, 0)),
        out_shape=jax.ShapeDtypeStruct((N_ATOMS, 128), jnp.float32),
    )(an2d, table_pad)


def _tc_pairs(dfx2, dfy2, dfz2, centers2d, widths2d, blk0, prev=None):
    rows = dfx2.shape[0]
    grid = (rows + _TC_R - 1) // _TC_R
    rows_all = N_PAIRS // 128

    def body(dx_ref, dy_ref, dz_ref, c_ref, w_ref, *rest):
        vec_ref, dist_ref, cut_ref, rbf_ref = rest[-4:]
        dx = dx_ref[...]
        dy = dy_ref[...]
        dz = dz_ref[...]
        d2 = dx * dx + dy * dy + dz * dz
        d = jnp.sqrt(d2)
        dist_ref[...] = d
        x = d * (1.0 / CUTOFF)
        x2 = x * x
        x3 = x2 * x
        poly = 1.0 - 6.0 * x3 * x2 + 15.0 * x2 * x2 - 10.0 * x3
        cut_ref[...] = jnp.where(x < 1.0, poly, 0.0)
        vec_ref[0:1, :] = dx.reshape(1, _TC_L)
        vec_ref[1:2, :] = dy.reshape(1, _TC_L)
        vec_ref[2:3, :] = dz.reshape(1, _TC_L)
        dl = d.reshape(1, _TC_L)
        c = c_ref[...]
        w = w_ref[...]
        coeff = -0.5 / (w * w)
        diff = dl - c
        rbf_ref[...] = jnp.exp(coeff * diff * diff)

    in_specs = [
        pl.BlockSpec((_TC_R, 128), lambda i: (i, 0)),
        pl.BlockSpec((_TC_R, 128), lambda i: (i, 0)),
        pl.BlockSpec((_TC_R, 128), lambda i: (i, 0)),
        pl.BlockSpec((N_RADIALBASIS, 1), lambda i: (0, 0)),
        pl.BlockSpec((N_RADIALBASIS, 1), lambda i: (0, 0)),
    ]
    args = [dfx2, dfy2, dfz2, centers2d, widths2d]
    aliases = {}
    if prev is not None:
        for k, arr in enumerate(prev):
            aliases[len(args)] = k
            args.append(arr)
            in_specs.append(pl.BlockSpec(memory_space=pl.ANY))
    return pl.pallas_call(
        body,
        grid=(grid,),
        in_specs=in_specs,
        out_specs=[
            pl.BlockSpec((3, _TC_L), lambda i: (0, i + blk0)),
            pl.BlockSpec((_TC_R, 128), lambda i: (i + blk0, 0)),
            pl.BlockSpec((_TC_R, 128), lambda i: (i + blk0, 0)),
            pl.BlockSpec((N_RADIALBASIS, _TC_L), lambda i: (0, i + blk0)),
        ],
        out_shape=[
            jax.ShapeDtypeStruct((3, N_PAIRS), jnp.float32),
            jax.ShapeDtypeStruct((rows_all, 128), jnp.float32),
            jax.ShapeDtypeStruct((rows_all, 128), jnp.float32),
            jax.ShapeDtypeStruct((N_RADIALBASIS, N_PAIRS), jnp.float32),
        ],
        input_output_aliases=aliases,
    )(*args)


_SPLITS = [0, 262144, 524288, N_PAIRS]


def kernel(atomic_numbers, positions, idx_i, idx_j, atom_features,
           rbf_centers, rbf_widths):
    pos8 = jnp.pad(positions, ((0, 0), (0, 5)))
    ii = idx_i.astype(jnp.int32)
    jj = idx_j.astype(jnp.int32)
    seg_planes = [
        _sc_pair_gather(pos8, ii, jj, lo, hi - lo)
        for lo, hi in zip(_SPLITS[:-1], _SPLITS[1:])
    ]
    table_pad = jnp.pad(atom_features, ((0, 128 - (N_MAXATOM + 1)), (0, 0)))
    features = _tc_embed(atomic_numbers.astype(jnp.int32).reshape(-1, 1),
                         table_pad)
    c2 = rbf_centers.reshape(-1, 1)
    w2 = rbf_widths.reshape(-1, 1)
    outs = None
    for lo, planes in zip(_SPLITS[:-1], seg_planes):
        outs = _tc_pairs(*(p.reshape(-1, 128) for p in planes), c2, w2,
                         lo // _TC_L, prev=outs)
    vecT, dist, cut, rbfT = outs
    return (features, dist.reshape(-1), vecT.T, cut.reshape(-1), rbfT.T)


# speedup vs baseline: 1.0168x; 1.0168x over previous
import functools

import jax
import jax.numpy as jnp
from jax import lax
from jax.experimental import pallas as pl
from jax.experimental.pallas import tpu as pltpu
from jax.experimental.pallas import tpu_sc as plsc

N_ATOMS = 50000
N_PAIRS = 800000
N_ATOMBASIS = 128
N_RADIALBASIS = 64
N_MAXATOM = 94
CUTOFF = 5.0

_SC_CHUNK = 1256
_SC_PAD = 1264
_TC_L = 16384
_TC_R = _TC_L // 128


def _sc_pair_gather(pos8, idx_i, idx_j, pair_lo, n_pairs):
    info = plsc.get_sparse_core_info()
    nc, ns = info.num_cores, info.num_subcores
    nw = nc * ns
    per_w = n_pairs // nw
    offs, szs = [], []
    o = 0
    while o < per_w:
        sz = min(_SC_CHUNK, per_w - o)
        offs.append(o)
        szs.append(sz)
        o += sz
    n_ch = len(offs)

    mesh = plsc.VectorSubcoreMesh(core_axis_name="c", subcore_axis_name="s")

    @functools.partial(
        pl.kernel,
        out_type=tuple(
            jax.ShapeDtypeStruct((n_pairs,), jnp.float32) for _ in range(3)
        ),
        mesh=mesh,
        scratch_types=[
            pltpu.VMEM((per_w,), jnp.int32),
            pltpu.VMEM((per_w,), jnp.int32),
            [pltpu.VMEM((_SC_PAD, 8), jnp.float32) for _ in range(2)],
            [pltpu.VMEM((_SC_PAD, 8), jnp.float32) for _ in range(2)],
            [pltpu.VMEM((_SC_PAD,), jnp.float32) for _ in range(2)],
            [pltpu.VMEM((_SC_PAD,), jnp.float32) for _ in range(2)],
            [pltpu.VMEM((_SC_PAD,), jnp.float32) for _ in range(2)],
            [pltpu.SemaphoreType.DMA for _ in range(2)],
            [pltpu.SemaphoreType.DMA for _ in range(2)],
            [pltpu.SemaphoreType.DMA for _ in range(2)],
        ],
        compiler_params=pltpu.CompilerParams(use_tc_tiling_on_sc=False,
                                             needs_layout_passes=False),
    )
    def gather_kernel(pos_hbm, ii_hbm, jj_hbm,
                      ox_hbm, oy_hbm, oz_hbm,
                      ii_v, jj_v, pi_v, pj_v,
                      dfx_v, dfy_v, dfz_v, sem_i, sem_j, sem_w):
        wid = lax.axis_index("s") * nc + lax.axis_index("c")
        lane = lax.iota(jnp.int32, 16)
        wbase = wid * per_w

        pltpu.sync_copy(ii_hbm.at[pl.ds(pair_lo + wbase, per_w)], ii_v)
        pltpu.sync_copy(jj_hbm.at[pl.ds(pair_lo + wbase, per_w)], jj_v)

        def run_compute(b, n_grp):
            @plsc.parallel_loop(0, n_grp * 16, 16, unroll=4)
            def _loop(base16):
                rows = base16 + lane
                c0 = jnp.zeros((16,), jnp.int32)
                ax = (plsc.load_gather(pj_v[b], [rows, c0])
                      - plsc.load_gather(pi_v[b], [rows, c0]))
                ay = (plsc.load_gather(pj_v[b], [rows, c0 + 1])
                      - plsc.load_gather(pi_v[b], [rows, c0 + 1]))
                az = (plsc.load_gather(pj_v[b], [rows, c0 + 2])
                      - plsc.load_gather(pi_v[b], [rows, c0 + 2]))
                dfx_v[b][pl.ds(base16, 16)] = ax
                dfy_v[b][pl.ds(base16, 16)] = ay
                dfz_v[b][pl.ds(base16, 16)] = az

        def start_chunk(t, b):
            sz = szs[t]
            cp_i = pltpu.async_copy(
                pos_hbm.at[ii_v.at[pl.ds(offs[t], sz)]],
                pi_v[b].at[pl.ds(0, sz)], sem_i[b])
            cp_j = pltpu.async_copy(
                pos_hbm.at[jj_v.at[pl.ds(offs[t], sz)]],
                pj_v[b].at[pl.ds(0, sz)], sem_j[b])
            return cp_i, cp_j

        cps = [start_chunk(0, 0),
               start_chunk(1, 1) if n_ch > 1 else None]
        wb = [None, None]
        for t in range(n_ch):
            b = t % 2
            cps[b][0].wait()
            cps[b][1].wait()
            if wb[b] is not None:
                for c in wb[b]:
                    c.wait()
            run_compute(b, (szs[t] + 15) // 16)
            base = wbase + offs[t]
            sz = szs[t]
            wb[b] = [
                pltpu.async_copy(src.at[pl.ds(0, sz)],
                                 dst.at[pl.ds(base, sz)], sem_w[b])
                for src, dst in ((dfx_v[b], ox_hbm), (dfy_v[b], oy_hbm),
                                 (dfz_v[b], oz_hbm))
            ]
            if t + 2 < n_ch:
                cps[b] = start_chunk(t + 2, b)
        for b in (0, 1):
            if wb[b] is not None:
                for c in wb[b]:
                    c.wait()

    return gather_kernel(pos8, idx_i, idx_j)


def _tc_embed(an2d, table_pad):
    blk = 2000

    def body(an_ref, tab_ref, out_ref):
        an = an_ref[...]
        oh = (lax.broadcasted_iota(jnp.int32, (blk, 128), 1) == an)
        emb = jnp.dot(oh.astype(jnp.float32), tab_ref[...],
                      preferred_element_type=jnp.float32)
        norm = jnp.sqrt(jnp.sum(emb * emb, axis=-1, keepdims=True))
        scale = jnp.minimum(1.0, float(N_ATOMBASIS) / (norm + 1e-7))
        out_ref[...] = emb * scale

    return pl.pallas_call(
        body,
        grid=(N_ATOMS // blk,),
        in_specs=[
            pl.BlockSpec((blk, 1), lambda i: (i, 0)),
            pl.BlockSpec((128, 128), lambda i: (0, 0)),
        ],
        out_specs=pl.BlockSpec((blk, 128), lambda i: (i, 0)),
        out_shape=jax.ShapeDtypeStruct((N_ATOMS, 128), jnp.float32),
    )(an2d, table_pad)


def _tc_pairs(dfx2, dfy2, dfz2, centers2d, widths2d, blk0, prev=None):
    rows = dfx2.shape[0]
    grid = (rows + _TC_R - 1) // _TC_R
    rows_all = N_PAIRS // 128

    def body(dx_ref, dy_ref, dz_ref, c_ref, w_ref, *rest):
        vec_ref, dist_ref, cut_ref, rbf_ref = rest[-4:]
        dx = dx_ref[...]
        dy = dy_ref[...]
        dz = dz_ref[...]
        d2 = dx * dx + dy * dy + dz * dz
        d = jnp.sqrt(d2)
        dist_ref[...] = d
        x = d * (1.0 / CUTOFF)
        x2 = x * x
        x3 = x2 * x
        poly = 1.0 - 6.0 * x3 * x2 + 15.0 * x2 * x2 - 10.0 * x3
        cut_ref[...] = jnp.where(x < 1.0, poly, 0.0)
        vec_ref[0:1, :] = dx.reshape(1, _TC_L)
        vec_ref[1:2, :] = dy.reshape(1, _TC_L)
        vec_ref[2:3, :] = dz.reshape(1, _TC_L)
        dl = d.reshape(1, _TC_L)
        c = c_ref[...]
        w = w_ref[...]
        coeff = -0.5 / (w * w)
        diff = dl - c
        rbf_ref[...] = jnp.exp(coeff * diff * diff)

    in_specs = [
        pl.BlockSpec((_TC_R, 128), lambda i: (i, 0)),
        pl.BlockSpec((_TC_R, 128), lambda i: (i, 0)),
        pl.BlockSpec((_TC_R, 128), lambda i: (i, 0)),
        pl.BlockSpec((N_RADIALBASIS, 1), lambda i: (0, 0)),
        pl.BlockSpec((N_RADIALBASIS, 1), lambda i: (0, 0)),
    ]
    args = [dfx2, dfy2, dfz2, centers2d, widths2d]
    aliases = {}
    if prev is not None:
        for k, arr in enumerate(prev):
            aliases[len(args)] = k
            args.append(arr)
            in_specs.append(pl.BlockSpec(memory_space=pl.ANY))
    return pl.pallas_call(
        body,
        grid=(grid,),
        in_specs=in_specs,
        out_specs=[
            pl.BlockSpec((3, _TC_L), lambda i: (0, i + blk0)),
            pl.BlockSpec((_TC_R, 128), lambda i: (i + blk0, 0)),
            pl.BlockSpec((_TC_R, 128), lambda i: (i + blk0, 0)),
            pl.BlockSpec((N_RADIALBASIS, _TC_L), lambda i: (0, i + blk0)),
        ],
        out_shape=[
            jax.ShapeDtypeStruct((3, N_PAIRS), jnp.float32),
            jax.ShapeDtypeStruct((rows_all, 128), jnp.float32),
            jax.ShapeDtypeStruct((rows_all, 128), jnp.float32),
            jax.ShapeDtypeStruct((N_RADIALBASIS, N_PAIRS), jnp.float32),
        ],
        input_output_aliases=aliases,
    )(*args)


_SPLITS = [0, 409600, N_PAIRS]


def kernel(atomic_numbers, positions, idx_i, idx_j, atom_features,
           rbf_centers, rbf_widths):
    pos8 = jnp.pad(positions, ((0, 0), (0, 5)))
    ii = idx_i.astype(jnp.int32)
    jj = idx_j.astype(jnp.int32)
    seg_planes = [
        _sc_pair_gather(pos8, ii, jj, lo, hi - lo)
        for lo, hi in zip(_SPLITS[:-1], _SPLITS[1:])
    ]
    table_pad = jnp.pad(atom_features, ((0, 128 - (N_MAXATOM + 1)), (0, 0)))
    features = _tc_embed(atomic_numbers.astype(jnp.int32).reshape(-1, 1),
                         table_pad)
    c2 = rbf_centers.reshape(-1, 1)
    w2 = rbf_widths.reshape(-1, 1)
    outs = None
    for lo, planes in zip(_SPLITS[:-1], seg_planes):
        outs = _tc_pairs(*(p.reshape(-1, 128) for p in planes), c2, w2,
                         lo // _TC_L, prev=outs)
    vecT, dist, cut, rbfT = outs
    return (features, dist.reshape(-1), vecT.T, cut.reshape(-1), rbfT.T)
